# Initial kernel scaffold; baseline (speedup 1.0000x reference)
import jax
import jax.numpy as jnp
from jax.experimental import pallas as pl


def kernel(x, W_enc, W_dec, b_enc, b_dec, gate_raw):
    raise NotImplementedError("write your pallas kernel here")


# speedup vs baseline: 9.1824x; 9.1824x over previous
import jax
import jax.numpy as jnp
import numpy as np
from jax import lax
from jax.experimental import pallas as pl
from jax.experimental.pallas import tpu as pltpu

_B = 16
_T = 16
_DIN = 2048
_DSAE = 8192
_K = 128

_NT = 1024
_KT = 1024

_INT_MIN = np.int32(-2147483648)


def _enc_body(x_ref, w_ref, b_ref, o_ref):
    o_ref[...] = (
        jnp.dot(x_ref[...], w_ref[...], preferred_element_type=jnp.float32,
                precision=lax.Precision.DEFAULT)
        + b_ref[...]
    )


def _sortable(pre):
    b = lax.bitcast_convert_type(pre, jnp.int32)
    return jnp.where(b >= 0, b, jnp.bitwise_xor(jnp.bitwise_not(b), _INT_MIN))


def _topk_mask(pre):
    s = _sortable(pre)

    def body(i, cur):
        bitval = lax.shift_left(jnp.int32(1), jnp.int32(31) - i)
        cand = cur + bitval
        cnt = jnp.sum((s >= cand).astype(jnp.int32), axis=1, keepdims=True)
        return jnp.where(cnt >= _K, cand, cur)

    cur0 = jnp.full((pre.shape[0], 1), _INT_MIN)
    cur = lax.fori_loop(0, 32, body, cur0)
    return s >= cur


def _rec_body(pre_ref, g_ref, z_ref):
    gate = 1.0 / (1.0 + jnp.exp(-g_ref[...]))
    z = jnp.zeros((_B, _DSAE), jnp.float32)
    for t in range(_T):
        pre = pre_ref[:, t * _DSAE:(t + 1) * _DSAE] + gate * z
        mask = _topk_mask(pre)
        z = jnp.where(mask, jnp.maximum(pre, 0.0), 0.0)
        z_ref[:, t * _DSAE:(t + 1) * _DSAE] = z


def _dec_body(z_ref, w_ref, b_ref, x_ref, xh_ref, loss_ref):
    k = pl.program_id(0)
    part = jnp.dot(z_ref[...], w_ref[...], preferred_element_type=jnp.float32,
                   precision=lax.Precision.DEFAULT)

    @pl.when(k == 0)
    def _():
        xh_ref[...] = part

    @pl.when(k > 0)
    def _():
        xh_ref[...] += part

    @pl.when(k == pl.num_programs(0) - 1)
    def _():
        xh = xh_ref[...] + b_ref[...]
        xh_ref[...] = xh
        d = xh - x_ref[...]
        loss_ref[0, 0] = jnp.sum(d * d) * (1.0 / (_B * _T))


def kernel(x, W_enc, W_dec, b_enc, b_dec, gate_raw):
    x2 = x.reshape(_B * _T, _DIN)

    pre = pl.pallas_call(
        _enc_body,
        grid=(_DSAE // _NT,),
        in_specs=[
            pl.BlockSpec((_B * _T, _DIN), lambda n: (0, 0)),
            pl.BlockSpec((_DIN, _NT), lambda n: (0, n)),
            pl.BlockSpec((1, _NT), lambda n: (0, n)),
        ],
        out_specs=pl.BlockSpec((_B * _T, _NT), lambda n: (0, n)),
        out_shape=jax.ShapeDtypeStruct((_B * _T, _DSAE), jnp.float32),
        compiler_params=pltpu.CompilerParams(
            dimension_semantics=("parallel",)),
    )(x2, W_enc, b_enc.reshape(1, _DSAE))

    pre_b = pre.reshape(_B, _T * _DSAE)

    z = pl.pallas_call(
        _rec_body,
        in_specs=[
            pl.BlockSpec((_B, _T * _DSAE), lambda: (0, 0)),
            pl.BlockSpec((1, _DSAE), lambda: (0, 0)),
        ],
        out_specs=pl.BlockSpec((_B, _T * _DSAE), lambda: (0, 0)),
        out_shape=jax.ShapeDtypeStruct((_B, _T * _DSAE), jnp.float32),
    )(pre_b, gate_raw.reshape(1, _DSAE))

    z2 = z.reshape(_B * _T, _DSAE)

    xh, loss = pl.pallas_call(
        _dec_body,
        grid=(_DSAE // _KT,),
        in_specs=[
            pl.BlockSpec((_B * _T, _KT), lambda k: (0, k)),
            pl.BlockSpec((_KT, _DIN), lambda k: (k, 0)),
            pl.BlockSpec((1, _DIN), lambda k: (0, 0)),
            pl.BlockSpec((_B * _T, _DIN), lambda k: (0, 0)),
        ],
        out_specs=[
            pl.BlockSpec((_B * _T, _DIN), lambda k: (0, 0)),
            pl.BlockSpec(memory_space=pltpu.SMEM, block_shape=(1, 1),
                         index_map=lambda k: (0, 0)),
        ],
        out_shape=[
            jax.ShapeDtypeStruct((_B * _T, _DIN), jnp.float32),
            jax.ShapeDtypeStruct((1, 1), jnp.float32),
        ],
        compiler_params=pltpu.CompilerParams(
            dimension_semantics=("arbitrary",)),
    )(z2, W_dec, b_dec.reshape(1, _DIN), x2)

    x_hat = xh.reshape(_B, _T, _DIN)
    z_last = z[:, (_T - 1) * _DSAE:]
    return (loss[0, 0], x_hat, z_last)


# speedup vs baseline: 9.5083x; 1.0355x over previous
import jax
import jax.numpy as jnp
import numpy as np
from jax import lax
from jax.experimental import pallas as pl
from jax.experimental.pallas import tpu as pltpu

_B = 16
_T = 16
_DIN = 2048
_DSAE = 8192
_K = 128

_NT = 1024
_KT = 1024

_INT_MIN = np.int32(-2147483648)


def _enc_body(x_ref, w_ref, b_ref, o_ref):
    o_ref[...] = (
        jnp.dot(x_ref[...], w_ref[...], preferred_element_type=jnp.float32,
                precision=lax.Precision.DEFAULT)
        + b_ref[...]
    )


def _sortable(pre):
    b = lax.bitcast_convert_type(pre, jnp.int32)
    return jnp.where(b >= 0, b, jnp.bitwise_xor(jnp.bitwise_not(b), _INT_MIN))


def _topk_mask(pre):
    s = _sortable(pre)
    rows = pre.shape[0]

    def cond(state):
        i, _, cnt_cur = state
        return jnp.logical_and(i < 32, jnp.any(cnt_cur != _K))

    def body(state):
        i, cur, cnt_cur = state
        bitval = lax.shift_left(jnp.int32(1), jnp.int32(31) - i)
        cand = cur + bitval
        cnt = jnp.sum((s >= cand).astype(jnp.int32), axis=1, keepdims=True)
        take = cnt >= _K
        return (i + 1, jnp.where(take, cand, cur),
                jnp.where(take, cnt, cnt_cur))

    state = (jnp.int32(0), jnp.full((rows, 1), _INT_MIN),
             jnp.full((rows, 1), jnp.int32(_DSAE)))
    _, cur, _ = lax.while_loop(cond, body, state)
    return s >= cur


def _rec_body(pre_ref, g_ref, z_ref):
    gate = 1.0 / (1.0 + jnp.exp(-g_ref[...]))
    z = jnp.zeros((_B, _DSAE), jnp.float32)
    for t in range(_T):
        pre = pre_ref[:, t * _DSAE:(t + 1) * _DSAE] + gate * z
        mask = _topk_mask(pre)
        z = jnp.where(mask, jnp.maximum(pre, 0.0), 0.0)
        z_ref[:, t * _DSAE:(t + 1) * _DSAE] = z


def _dec_body(z_ref, w_ref, b_ref, x_ref, xh_ref, loss_ref):
    k = pl.program_id(0)
    part = jnp.dot(z_ref[...], w_ref[...], preferred_element_type=jnp.float32,
                   precision=lax.Precision.DEFAULT)

    @pl.when(k == 0)
    def _():
        xh_ref[...] = part

    @pl.when(k > 0)
    def _():
        xh_ref[...] += part

    @pl.when(k == pl.num_programs(0) - 1)
    def _():
        xh = xh_ref[...] + b_ref[...]
        xh_ref[...] = xh
        d = xh - x_ref[...]
        loss_ref[0, 0] = jnp.sum(d * d) * (1.0 / (_B * _T))


def kernel(x, W_enc, W_dec, b_enc, b_dec, gate_raw):
    x2 = x.reshape(_B * _T, _DIN)

    pre = pl.pallas_call(
        _enc_body,
        grid=(_DSAE // _NT,),
        in_specs=[
            pl.BlockSpec((_B * _T, _DIN), lambda n: (0, 0)),
            pl.BlockSpec((_DIN, _NT), lambda n: (0, n)),
            pl.BlockSpec((1, _NT), lambda n: (0, n)),
        ],
        out_specs=pl.BlockSpec((_B * _T, _NT), lambda n: (0, n)),
        out_shape=jax.ShapeDtypeStruct((_B * _T, _DSAE), jnp.float32),
        compiler_params=pltpu.CompilerParams(
            dimension_semantics=("parallel",)),
    )(x2, W_enc, b_enc.reshape(1, _DSAE))

    pre_b = pre.reshape(_B, _T * _DSAE)

    z = pl.pallas_call(
        _rec_body,
        in_specs=[
            pl.BlockSpec((_B, _T * _DSAE), lambda: (0, 0)),
            pl.BlockSpec((1, _DSAE), lambda: (0, 0)),
        ],
        out_specs=pl.BlockSpec((_B, _T * _DSAE), lambda: (0, 0)),
        out_shape=jax.ShapeDtypeStruct((_B, _T * _DSAE), jnp.float32),
    )(pre_b, gate_raw.reshape(1, _DSAE))

    z2 = z.reshape(_B * _T, _DSAE)

    xh, loss = pl.pallas_call(
        _dec_body,
        grid=(_DSAE // _KT,),
        in_specs=[
            pl.BlockSpec((_B * _T, _KT), lambda k: (0, k)),
            pl.BlockSpec((_KT, _DIN), lambda k: (k, 0)),
            pl.BlockSpec((1, _DIN), lambda k: (0, 0)),
            pl.BlockSpec((_B * _T, _DIN), lambda k: (0, 0)),
        ],
        out_specs=[
            pl.BlockSpec((_B * _T, _DIN), lambda k: (0, 0)),
            pl.BlockSpec(memory_space=pltpu.SMEM, block_shape=(1, 1),
                         index_map=lambda k: (0, 0)),
        ],
        out_shape=[
            jax.ShapeDtypeStruct((_B * _T, _DIN), jnp.float32),
            jax.ShapeDtypeStruct((1, 1), jnp.float32),
        ],
        compiler_params=pltpu.CompilerParams(
            dimension_semantics=("arbitrary",)),
    )(z2, W_dec, b_dec.reshape(1, _DIN), x2)

    x_hat = xh.reshape(_B, _T, _DIN)
    z_last = z[:, (_T - 1) * _DSAE:]
    return (loss[0, 0], x_hat, z_last)


# speedup vs baseline: 9.7343x; 1.0238x over previous
import jax
import jax.numpy as jnp
import numpy as np
from jax import lax
from jax.experimental import pallas as pl
from jax.experimental.pallas import tpu as pltpu

_B = 16
_T = 16
_DIN = 2048
_DSAE = 8192
_K = 128

_NT = 1024
_KT = 1024

_INT_MIN = np.int32(-2147483648)


def _enc_body(x_ref, w_ref, b_ref, o_ref):
    o_ref[...] = (
        jnp.dot(x_ref[...], w_ref[...], preferred_element_type=jnp.float32,
                precision=lax.Precision.DEFAULT)
        + b_ref[...]
    )


def _sortable(pre):
    b = lax.bitcast_convert_type(pre, jnp.int32)
    return jnp.where(b >= 0, b, jnp.bitwise_xor(jnp.bitwise_not(b), _INT_MIN))


def _topk_mask(pre):
    s = _sortable(pre)
    rows = pre.shape[0]

    def body(i, cur):
        step = lax.shift_left(jnp.int32(1), jnp.int32(30) - 2 * i)
        c1 = cur + step
        c2 = c1 + step
        c3 = c2 + step
        n1 = jnp.sum((s >= c1).astype(jnp.int32), axis=1, keepdims=True)
        n2 = jnp.sum((s >= c2).astype(jnp.int32), axis=1, keepdims=True)
        n3 = jnp.sum((s >= c3).astype(jnp.int32), axis=1, keepdims=True)
        m = ((n1 >= _K).astype(jnp.int32) + (n2 >= _K).astype(jnp.int32)
             + (n3 >= _K).astype(jnp.int32))
        return cur + m * step

    cur0 = jnp.full((rows, 1), _INT_MIN)
    cur = lax.fori_loop(0, 16, body, cur0)
    return s >= cur


def _rec_body(pre_ref, g_ref, z_ref):
    gate = 1.0 / (1.0 + jnp.exp(-g_ref[...]))
    z = jnp.zeros((_B, _DSAE), jnp.float32)
    for t in range(_T):
        pre = pre_ref[:, t * _DSAE:(t + 1) * _DSAE] + gate * z
        mask = _topk_mask(pre)
        z = jnp.where(mask, jnp.maximum(pre, 0.0), 0.0)
        z_ref[:, t * _DSAE:(t + 1) * _DSAE] = z


def _dec_body(z_ref, w_ref, b_ref, x_ref, xh_ref, loss_ref):
    k = pl.program_id(0)
    part = jnp.dot(z_ref[...], w_ref[...], preferred_element_type=jnp.float32,
                   precision=lax.Precision.DEFAULT)

    @pl.when(k == 0)
    def _():
        xh_ref[...] = part

    @pl.when(k > 0)
    def _():
        xh_ref[...] += part

    @pl.when(k == pl.num_programs(0) - 1)
    def _():
        xh = xh_ref[...] + b_ref[...]
        xh_ref[...] = xh
        d = xh - x_ref[...]
        loss_ref[0, 0] = jnp.sum(d * d) * (1.0 / (_B * _T))


def kernel(x, W_enc, W_dec, b_enc, b_dec, gate_raw):
    x2 = x.reshape(_B * _T, _DIN)

    pre = pl.pallas_call(
        _enc_body,
        grid=(_DSAE // _NT,),
        in_specs=[
            pl.BlockSpec((_B * _T, _DIN), lambda n: (0, 0)),
            pl.BlockSpec((_DIN, _NT), lambda n: (0, n)),
            pl.BlockSpec((1, _NT), lambda n: (0, n)),
        ],
        out_specs=pl.BlockSpec((_B * _T, _NT), lambda n: (0, n)),
        out_shape=jax.ShapeDtypeStruct((_B * _T, _DSAE), jnp.float32),
        compiler_params=pltpu.CompilerParams(
            dimension_semantics=("parallel",)),
    )(x2, W_enc, b_enc.reshape(1, _DSAE))

    pre_b = pre.reshape(_B, _T * _DSAE)

    z = pl.pallas_call(
        _rec_body,
        in_specs=[
            pl.BlockSpec((_B, _T * _DSAE), lambda: (0, 0)),
            pl.BlockSpec((1, _DSAE), lambda: (0, 0)),
        ],
        out_specs=pl.BlockSpec((_B, _T * _DSAE), lambda: (0, 0)),
        out_shape=jax.ShapeDtypeStruct((_B, _T * _DSAE), jnp.float32),
    )(pre_b, gate_raw.reshape(1, _DSAE))

    z2 = z.reshape(_B * _T, _DSAE)

    xh, loss = pl.pallas_call(
        _dec_body,
        grid=(_DSAE // _KT,),
        in_specs=[
            pl.BlockSpec((_B * _T, _KT), lambda k: (0, k)),
            pl.BlockSpec((_KT, _DIN), lambda k: (k, 0)),
            pl.BlockSpec((1, _DIN), lambda k: (0, 0)),
            pl.BlockSpec((_B * _T, _DIN), lambda k: (0, 0)),
        ],
        out_specs=[
            pl.BlockSpec((_B * _T, _DIN), lambda k: (0, 0)),
            pl.BlockSpec(memory_space=pltpu.SMEM, block_shape=(1, 1),
                         index_map=lambda k: (0, 0)),
        ],
        out_shape=[
            jax.ShapeDtypeStruct((_B * _T, _DIN), jnp.float32),
            jax.ShapeDtypeStruct((1, 1), jnp.float32),
        ],
        compiler_params=pltpu.CompilerParams(
            dimension_semantics=("arbitrary",)),
    )(z2, W_dec, b_dec.reshape(1, _DIN), x2)

    x_hat = xh.reshape(_B, _T, _DIN)
    z_last = z[:, (_T - 1) * _DSAE:]
    return (loss[0, 0], x_hat, z_last)


# speedup vs baseline: 9.7978x; 1.0065x over previous
import jax
import jax.numpy as jnp
import numpy as np
from jax import lax
from jax.experimental import pallas as pl
from jax.experimental.pallas import tpu as pltpu

_B = 16
_T = 16
_DIN = 2048
_DSAE = 8192
_K = 128

_NT = 1024
_KT = 1024

_INT_MIN = np.int32(-2147483648)


def _enc_body(x_ref, w_ref, b_ref, o_ref):
    o_ref[...] = (
        jnp.dot(x_ref[...], w_ref[...], preferred_element_type=jnp.float32,
                precision=lax.Precision.DEFAULT)
        + b_ref[...]
    )


def _unsort_f(sv):
    bits = jnp.where(sv >= 0, sv, jnp.bitwise_not(jnp.bitwise_xor(sv, _INT_MIN)))
    return lax.bitcast_convert_type(bits, jnp.float32)


_CH = 16
_W = _DSAE // _CH


def _rec_body(pre_ref, g_ref, z_ref, pbuf, gbuf):
    gbuf[...] = 1.0 / (1.0 + jnp.exp(-g_ref[...]))
    for t in range(_T):
        base = t * _DSAE
        pbase = (t - 1) * _DSAE
        for c in range(_CH):
            lo = c * _W
            p = pre_ref[:, base + lo:base + lo + _W]
            if t > 0:
                p = p + gbuf[:, lo:lo + _W] * z_ref[:, pbase + lo:pbase + lo + _W]
            pbuf[:, lo:lo + _W] = p

        def rnd(i, cur):
            step = lax.shift_left(jnp.int32(1), jnp.int32(30) - 2 * i)
            c1 = cur + step
            c2 = c1 + step
            c3 = c2 + step
            t1 = _unsort_f(c1)
            t2 = _unsort_f(c2)
            t3 = _unsort_f(c3)
            a1 = jnp.zeros((_B, _W), jnp.int32)
            a2 = jnp.zeros((_B, _W), jnp.int32)
            a3 = jnp.zeros((_B, _W), jnp.int32)
            for c in range(_CH):
                blk = pbuf[:, c * _W:(c + 1) * _W]
                a1 = a1 + (blk >= t1).astype(jnp.int32)
                a2 = a2 + (blk >= t2).astype(jnp.int32)
                a3 = a3 + (blk >= t3).astype(jnp.int32)
            n1 = jnp.sum(a1, axis=1, keepdims=True)
            n2 = jnp.sum(a2, axis=1, keepdims=True)
            n3 = jnp.sum(a3, axis=1, keepdims=True)
            m = ((n1 >= _K).astype(jnp.int32) + (n2 >= _K).astype(jnp.int32)
                 + (n3 >= _K).astype(jnp.int32))
            return cur + m * step

        cur = lax.fori_loop(0, 16, rnd, jnp.full((_B, 1), _INT_MIN))
        thr = _unsort_f(cur)
        for c in range(_CH):
            lo = c * _W
            p = pbuf[:, lo:lo + _W]
            z_ref[:, base + lo:base + lo + _W] = jnp.where(
                p >= thr, jnp.maximum(p, 0.0), 0.0)


def _dec_body(z_ref, w_ref, b_ref, x_ref, xh_ref, loss_ref):
    k = pl.program_id(0)
    part = jnp.dot(z_ref[...], w_ref[...], preferred_element_type=jnp.float32,
                   precision=lax.Precision.DEFAULT)

    @pl.when(k == 0)
    def _():
        xh_ref[...] = part

    @pl.when(k > 0)
    def _():
        xh_ref[...] += part

    @pl.when(k == pl.num_programs(0) - 1)
    def _():
        xh = xh_ref[...] + b_ref[...]
        xh_ref[...] = xh
        d = xh - x_ref[...]
        loss_ref[0, 0] = jnp.sum(d * d) * (1.0 / (_B * _T))


def kernel(x, W_enc, W_dec, b_enc, b_dec, gate_raw):
    x2 = x.reshape(_B * _T, _DIN)

    pre = pl.pallas_call(
        _enc_body,
        grid=(_DSAE // _NT,),
        in_specs=[
            pl.BlockSpec((_B * _T, _DIN), lambda n: (0, 0)),
            pl.BlockSpec((_DIN, _NT), lambda n: (0, n)),
            pl.BlockSpec((1, _NT), lambda n: (0, n)),
        ],
        out_specs=pl.BlockSpec((_B * _T, _NT), lambda n: (0, n)),
        out_shape=jax.ShapeDtypeStruct((_B * _T, _DSAE), jnp.float32),
        compiler_params=pltpu.CompilerParams(
            dimension_semantics=("parallel",)),
    )(x2, W_enc, b_enc.reshape(1, _DSAE))

    pre_b = pre.reshape(_B, _T * _DSAE)

    z = pl.pallas_call(
        _rec_body,
        in_specs=[
            pl.BlockSpec((_B, _T * _DSAE), lambda: (0, 0)),
            pl.BlockSpec((1, _DSAE), lambda: (0, 0)),
        ],
        out_specs=pl.BlockSpec((_B, _T * _DSAE), lambda: (0, 0)),
        out_shape=jax.ShapeDtypeStruct((_B, _T * _DSAE), jnp.float32),
        scratch_shapes=[
            pltpu.VMEM((_B, _DSAE), jnp.float32),
            pltpu.VMEM((1, _DSAE), jnp.float32),
        ],
    )(pre_b, gate_raw.reshape(1, _DSAE))

    z2 = z.reshape(_B * _T, _DSAE)

    xh, loss = pl.pallas_call(
        _dec_body,
        grid=(_DSAE // _KT,),
        in_specs=[
            pl.BlockSpec((_B * _T, _KT), lambda k: (0, k)),
            pl.BlockSpec((_KT, _DIN), lambda k: (k, 0)),
            pl.BlockSpec((1, _DIN), lambda k: (0, 0)),
            pl.BlockSpec((_B * _T, _DIN), lambda k: (0, 0)),
        ],
        out_specs=[
            pl.BlockSpec((_B * _T, _DIN), lambda k: (0, 0)),
            pl.BlockSpec(memory_space=pltpu.SMEM, block_shape=(1, 1),
                         index_map=lambda k: (0, 0)),
        ],
        out_shape=[
            jax.ShapeDtypeStruct((_B * _T, _DIN), jnp.float32),
            jax.ShapeDtypeStruct((1, 1), jnp.float32),
        ],
        compiler_params=pltpu.CompilerParams(
            dimension_semantics=("arbitrary",)),
    )(z2, W_dec, b_dec.reshape(1, _DIN), x2)

    x_hat = xh.reshape(_B, _T, _DIN)
    z_last = z[:, (_T - 1) * _DSAE:]
    return (loss[0, 0], x_hat, z_last)
